# baseline (speedup 1.0000x reference)
import functools
import jax
import jax.numpy as jnp
from jax import lax
from jax.experimental import pallas as pl
from jax.experimental.pallas import tpu as pltpu, tpu_sc as plsc

SEQ = 2048
D = 2048
CHUNK_ROWS = 16
NBUF = 3
LANES = 16
TC_BLOCK_ROWS = 256


def _make_sc_kernel(nc, ns):
    nw = nc * ns
    rows_per_worker = SEQ // nw
    n_chunks = rows_per_worker // CHUNK_ROWS

    mesh = plsc.VectorSubcoreMesh(core_axis_name="c", subcore_axis_name="s")

    @functools.partial(
        pl.kernel,
        out_type=jax.ShapeDtypeStruct((SEQ, D), jnp.float32),
        mesh=mesh,
        scratch_types=(
            pltpu.VMEM((LANES,), jnp.int32),
            [pltpu.VMEM((CHUNK_ROWS, D), jnp.float32) for _ in range(NBUF)],
            [pltpu.SemaphoreType.DMA for _ in range(NBUF)],
            [pltpu.SemaphoreType.DMA for _ in range(NBUF)],
            [pltpu.SemaphoreType.DMA for _ in range(NBUF)],
        ),
    )
    def sc_kernel(feat_hbm, tab_hbm, out_hbm, zidx, bufs,
                  in_sems, add_sems, out_sems):
        cid = lax.axis_index("c")
        sid = lax.axis_index("s")
        wid = sid * nc + cid
        base_row = wid * rows_per_worker

        zidx[...] = jnp.zeros((LANES,), jnp.int32)

        rows = [base_row + ci * CHUNK_ROWS for ci in range(n_chunks)]
        nk = len(rows)

        def start_in(k):
            return pltpu.async_copy(
                tab_hbm.at[pl.ds(rows[k], CHUNK_ROWS)], bufs[k % NBUF],
                in_sems[k % NBUF])

        def start_add(k):
            return pltpu.async_copy(
                feat_hbm.at[zidx], bufs[k % NBUF], add_sems[k % NBUF],
                add=True)

        def start_out(k):
            return pltpu.async_copy(
                bufs[k % NBUF], out_hbm.at[pl.ds(rows[k], CHUNK_ROWS)],
                out_sems[k % NBUF])

        in_fly = {0: start_in(0), 1: start_in(1)}
        out_fly = {}
        for k in range(nk):
            nxt = k + 2
            if nxt < nk:
                prev = nxt - NBUF
                if prev >= 0:
                    out_fly[prev].wait()
                in_fly[nxt] = start_in(nxt)
            in_fly[k].wait()
            start_add(k).wait()
            out_fly[k] = start_out(k)
        for k in range(max(0, nk - NBUF), nk):
            out_fly[k].wait()

    return sc_kernel


def _tc_add_kernel(feat_ref, tab_ref, out_ref):
    out_ref[...] = tab_ref[...] + feat_ref[...]


def _tc_add(feat2d, table):
    return pl.pallas_call(
        _tc_add_kernel,
        grid=(SEQ // TC_BLOCK_ROWS,),
        in_specs=[
            pl.BlockSpec((1, D), lambda i: (0, 0)),
            pl.BlockSpec((TC_BLOCK_ROWS, D), lambda i: (i, 0)),
        ],
        out_specs=pl.BlockSpec((TC_BLOCK_ROWS, D), lambda i: (i, 0)),
        out_shape=jax.ShapeDtypeStruct((SEQ, D), jnp.float32),
    )(feat2d, table)


def kernel(text, image, pos_table, text_pos_table, image_pos_table):
    del pos_table
    info = plsc.get_sparse_core_info()
    sc_k = _make_sc_kernel(info.num_cores, info.num_subcores)

    iout = sc_k(image, image_pos_table)
    tout = _tc_add(text, text_pos_table)
    return (tout[None], iout[None])


# speedup vs baseline: 3.3362x; 3.3362x over previous
import functools
import jax
import jax.numpy as jnp
from jax import lax
from jax.experimental import pallas as pl
from jax.experimental.pallas import tpu as pltpu, tpu_sc as plsc

SEQ = 2048
D = 2048
CHUNK_ROWS = 16
NBUF = 3
LANES = 16
GROUP = 32
TC_BLOCK_ROWS = 512


def _make_sc_kernel(nc, ns):
    nw = nc * ns
    rows_per_worker = SEQ // nw
    n_chunks = rows_per_worker // CHUNK_ROWS
    n_groups = D // (GROUP * LANES)

    mesh = plsc.VectorSubcoreMesh(core_axis_name="c", subcore_axis_name="s")

    @functools.partial(
        pl.kernel,
        out_type=jax.ShapeDtypeStruct((SEQ, D), jnp.float32),
        mesh=mesh,
        scratch_types=(
            pltpu.VMEM((D,), jnp.float32),
            [pltpu.VMEM((CHUNK_ROWS, D), jnp.float32) for _ in range(NBUF)],
            [pltpu.SemaphoreType.DMA for _ in range(NBUF)],
            [pltpu.SemaphoreType.DMA for _ in range(NBUF)],
        ),
    )
    def sc_kernel(feat_hbm, tab_hbm, out_hbm, feat_v, bufs, in_sems, out_sems):
        wid = lax.axis_index("s") * nc + lax.axis_index("c")
        base_row = wid * rows_per_worker

        pltpu.sync_copy(feat_hbm, feat_v)

        rows = [base_row + ci * CHUNK_ROWS for ci in range(n_chunks)]
        nk = len(rows)

        def start_in(k):
            return pltpu.async_copy(
                tab_hbm.at[pl.ds(rows[k], CHUNK_ROWS)], bufs[k % NBUF],
                in_sems[k % NBUF])

        def start_out(k):
            return pltpu.async_copy(
                bufs[k % NBUF], out_hbm.at[pl.ds(rows[k], CHUNK_ROWS)],
                out_sems[k % NBUF])

        def compute(k):
            buf = bufs[k % NBUF]

            def gbody(g, _):
                base_col = g * GROUP * LANES
                fj = [feat_v[pl.ds(base_col + c * LANES, LANES)]
                      for c in range(GROUP)]

                @plsc.parallel_loop(0, CHUNK_ROWS, step=1, unroll=4)
                def rbody(r):
                    for c in range(GROUP):
                        sl = pl.ds(base_col + c * LANES, LANES)
                        buf[r, sl] = buf[r, sl] + fj[c]

                return 0

            lax.fori_loop(0, n_groups, gbody, 0)

        in_fly = {0: start_in(0), 1: start_in(1)}
        out_fly = {}
        for k in range(nk):
            nxt = k + 2
            if nxt < nk:
                prev = nxt - NBUF
                if prev >= 0:
                    out_fly[prev].wait()
                in_fly[nxt] = start_in(nxt)
            in_fly[k].wait()
            compute(k)
            out_fly[k] = start_out(k)
        for k in range(max(0, nk - NBUF), nk):
            out_fly[k].wait()

    return sc_kernel


def _tc_add_kernel(feat_ref, tab_ref, out_ref):
    out_ref[...] = tab_ref[...] + feat_ref[...]


def _tc_add(feat2d, table):
    return pl.pallas_call(
        _tc_add_kernel,
        grid=(SEQ // TC_BLOCK_ROWS,),
        in_specs=[
            pl.BlockSpec((1, D), lambda i: (0, 0)),
            pl.BlockSpec((TC_BLOCK_ROWS, D), lambda i: (i, 0)),
        ],
        out_specs=pl.BlockSpec((TC_BLOCK_ROWS, D), lambda i: (i, 0)),
        out_shape=jax.ShapeDtypeStruct((SEQ, D), jnp.float32),
    )(feat2d, table)


def kernel(text, image, pos_table, text_pos_table, image_pos_table):
    del pos_table
    info = plsc.get_sparse_core_info()
    sc_k = _make_sc_kernel(info.num_cores, info.num_subcores)

    tout = _tc_add(text, text_pos_table)
    iout = sc_k(image.reshape(-1), image_pos_table)
    return (tout[None], iout[None])


# speedup vs baseline: 3.3938x; 1.0172x over previous
import functools
import jax
import jax.numpy as jnp
from jax import lax
from jax.experimental import pallas as pl
from jax.experimental.pallas import tpu as pltpu, tpu_sc as plsc

SEQ = 2048
D = 2048
CHUNK_ROWS = 16
NBUF = 3
LANES = 16
GROUP = 32
TC_BLOCK_ROWS = 512


def _make_sc_kernel(nc, ns):
    nw = nc * ns
    rows_per_worker = SEQ // nw
    n_chunks = rows_per_worker // CHUNK_ROWS
    n_groups = D // (GROUP * LANES)

    mesh = plsc.VectorSubcoreMesh(core_axis_name="c", subcore_axis_name="s")

    @functools.partial(
        pl.kernel,
        out_type=jax.ShapeDtypeStruct((SEQ, D), jnp.float32),
        mesh=mesh,
        scratch_types=(
            pltpu.VMEM((D,), jnp.float32),
            [pltpu.VMEM((CHUNK_ROWS, D), jnp.float32) for _ in range(NBUF)],
            [pltpu.SemaphoreType.DMA for _ in range(NBUF)],
            [pltpu.SemaphoreType.DMA for _ in range(NBUF)],
        ),
    )
    def sc_kernel(feat_hbm, tab_hbm, out_hbm, feat_v, bufs, in_sems, out_sems):
        wid = lax.axis_index("s") * nc + lax.axis_index("c")
        base_row = wid * rows_per_worker

        rows = [base_row + ci * CHUNK_ROWS for ci in range(n_chunks)]
        nk = len(rows)

        def start_in(k):
            return pltpu.async_copy(
                tab_hbm.at[pl.ds(rows[k], CHUNK_ROWS)], bufs[k % NBUF],
                in_sems[k % NBUF])

        def start_out(k):
            return pltpu.async_copy(
                bufs[k % NBUF], out_hbm.at[pl.ds(rows[k], CHUNK_ROWS)],
                out_sems[k % NBUF])

        def compute(k):
            buf = bufs[k % NBUF]

            def gbody(g, _):
                base_col = g * GROUP * LANES
                fj = [feat_v[pl.ds(base_col + c * LANES, LANES)]
                      for c in range(GROUP)]

                @plsc.parallel_loop(0, CHUNK_ROWS, step=1, unroll=4)
                def rbody(r):
                    for c in range(GROUP):
                        sl = pl.ds(base_col + c * LANES, LANES)
                        buf[r, sl] = buf[r, sl] + fj[c]

                return 0

            lax.fori_loop(0, n_groups, gbody, 0)

        in_fly = {0: start_in(0), 1: start_in(1)}
        pltpu.sync_copy(feat_hbm, feat_v)
        out_fly = {}
        for k in range(nk):
            nxt = k + 2
            if nxt < nk:
                prev = nxt - NBUF
                if prev >= 0:
                    out_fly[prev].wait()
                in_fly[nxt] = start_in(nxt)
            in_fly[k].wait()
            compute(k)
            out_fly[k] = start_out(k)
        for k in range(max(0, nk - NBUF), nk):
            out_fly[k].wait()

    return sc_kernel


def _tc_add_kernel(feat_ref, tab_ref, out_ref):
    out_ref[...] = tab_ref[...] + feat_ref[...]


def _tc_add(feat2d, table):
    return pl.pallas_call(
        _tc_add_kernel,
        grid=(SEQ // TC_BLOCK_ROWS,),
        in_specs=[
            pl.BlockSpec((1, D), lambda i: (0, 0)),
            pl.BlockSpec((TC_BLOCK_ROWS, D), lambda i: (i, 0)),
        ],
        out_specs=pl.BlockSpec((TC_BLOCK_ROWS, D), lambda i: (i, 0)),
        out_shape=jax.ShapeDtypeStruct((SEQ, D), jnp.float32),
    )(feat2d, table)


def kernel(text, image, pos_table, text_pos_table, image_pos_table):
    del pos_table
    info = plsc.get_sparse_core_info()
    sc_k = _make_sc_kernel(info.num_cores, info.num_subcores)

    tout = _tc_add(text, text_pos_table)
    iout = sc_k(image.reshape(-1), image_pos_table)
    return (tout[None], iout[None])
